# Initial kernel scaffold; baseline (speedup 1.0000x reference)
import jax
import jax.numpy as jnp
from jax.experimental import pallas as pl


def kernel(hidden_states, enc_w1, enc_b1, ln_g, ln_b, enc_w2, enc_b2, in_proj_w, in_proj_b, out_proj_w, out_proj_b, router_w):
    raise NotImplementedError("write your pallas kernel here")


# speedup vs baseline: 3.2982x; 3.2982x over previous
import jax
import jax.numpy as jnp
from jax import lax
from jax.experimental import pallas as pl
from jax.experimental.pallas import tpu as pltpu

_H = 768
_E = 64
_B = 32768
_BLK = 2048


def _dot(a, b):
    return lax.dot_general(a, b, (((1,), (0,)), ((), ())),
                           preferred_element_type=jnp.float32)


def _main_body(x_ref, wvt_ref, wot_ref, rw1t_ref, rw2t_ref, bv_ref, bo_ref,
               logits_ref, idx_ref, w_ref, lv_ref, ent_ref,
               load_acc, ent_acc):
    i = pl.program_id(0)
    nsteps = pl.num_programs(0)

    @pl.when(i == 0)
    def _init():
        load_acc[...] = jnp.zeros_like(load_acc)
        ent_acc[...] = jnp.zeros_like(ent_acc)

    x16 = x_ref[...].astype(jnp.bfloat16)
    v = _dot(x16, wvt_ref[...]) + bv_ref[...]
    a = _dot(v.astype(jnp.bfloat16), wot_ref[...]) + bo_ref[...]
    logits = _dot(x16, rw1t_ref[...]) + _dot(a.astype(jnp.bfloat16),
                                             rw2t_ref[...])
    logits_ref[...] = logits

    m1 = jnp.max(logits, axis=-1, keepdims=True)
    e = jnp.exp(logits - m1)
    s = jnp.sum(e, axis=-1, keepdims=True)
    p = e / s

    load_acc[...] += jnp.sum(p, axis=0, keepdims=True)
    ent_acc[...] += jnp.sum(p * jnp.log(p + 1e-8), axis=0, keepdims=True)

    col = lax.broadcasted_iota(jnp.int32, logits.shape, 1)
    idx1 = jnp.min(jnp.where(logits == m1, col, _E), axis=-1, keepdims=True)
    masked = jnp.where(col == idx1, -jnp.inf, logits)
    m2 = jnp.max(masked, axis=-1, keepdims=True)
    idx2 = jnp.min(jnp.where(masked == m2, col, _E), axis=-1, keepdims=True)
    idx_ref[...] = jnp.concatenate([idx1, idx2], axis=1)

    t = jnp.exp(m2 - m1)
    w1 = 1.0 / (1.0 + t)
    w_ref[...] = jnp.concatenate([w1, 1.0 - w1], axis=1)

    @pl.when(i == nsteps - 1)
    def _finalize():
        el = load_acc[...] / _B
        mu = jnp.mean(el)
        lv_ref[...] = jnp.sum((el - mu) ** 2, keepdims=True)[:, :1] / (_E - 1)
        ent_ref[...] = -jnp.sum(ent_acc[...], keepdims=True)[:, :1] / _B


def kernel(hidden_states, enc_w1, enc_b1, ln_g, ln_b, enc_w2, enc_b2,
           in_proj_w, in_proj_b, out_proj_w, out_proj_b, router_w):
    wvt = in_proj_w[2 * _H:].T.astype(jnp.bfloat16)
    wot = out_proj_w.T.astype(jnp.bfloat16)
    rw1t = router_w[:, :_H].T.astype(jnp.bfloat16)
    rw2t = router_w[:, _H:].T.astype(jnp.bfloat16)
    bv = in_proj_b[2 * _H:].reshape(1, _H)
    bo = out_proj_b.reshape(1, _H)

    nblk = _B // _BLK
    logits, idx, w, lv, ent = pl.pallas_call(
        _main_body,
        grid=(nblk,),
        in_specs=[
            pl.BlockSpec((_BLK, _H), lambda i: (i, 0)),
            pl.BlockSpec((_H, _H), lambda i: (0, 0)),
            pl.BlockSpec((_H, _H), lambda i: (0, 0)),
            pl.BlockSpec((_H, _E), lambda i: (0, 0)),
            pl.BlockSpec((_H, _E), lambda i: (0, 0)),
            pl.BlockSpec((1, _H), lambda i: (0, 0)),
            pl.BlockSpec((1, _H), lambda i: (0, 0)),
        ],
        out_specs=[
            pl.BlockSpec((_BLK, _E), lambda i: (i, 0)),
            pl.BlockSpec((_BLK, 2), lambda i: (i, 0)),
            pl.BlockSpec((_BLK, 2), lambda i: (i, 0)),
            pl.BlockSpec((1, 1), lambda i: (0, 0)),
            pl.BlockSpec((1, 1), lambda i: (0, 0)),
        ],
        out_shape=[
            jax.ShapeDtypeStruct((_B, _E), jnp.float32),
            jax.ShapeDtypeStruct((_B, 2), jnp.int32),
            jax.ShapeDtypeStruct((_B, 2), jnp.float32),
            jax.ShapeDtypeStruct((1, 1), jnp.float32),
            jax.ShapeDtypeStruct((1, 1), jnp.float32),
        ],
        scratch_shapes=[
            pltpu.VMEM((1, _E), jnp.float32),
            pltpu.VMEM((1, _E), jnp.float32),
        ],
        compiler_params=pltpu.CompilerParams(
            dimension_semantics=("arbitrary",)),
    )(hidden_states, wvt, wot, rw1t, rw2t, bv, bo)

    return (logits, idx, w, lv.reshape(()), ent.reshape(()))
